# Initial kernel scaffold; baseline (speedup 1.0000x reference)
import jax
import jax.numpy as jnp
from jax.experimental import pallas as pl


def kernel(x, diag1, w2, diag3, diff_indices, i_idx, j_idx):
    raise NotImplementedError("write your pallas kernel here")


# speedup vs baseline: 4.2875x; 4.2875x over previous
import functools

import jax
import jax.numpy as jnp
from jax import lax
from jax.experimental import pallas as pl
from jax.experimental.pallas import tpu as pltpu
from jax.experimental.pallas import tpu_sc as plsc

_C = 8
_PAIRS = [(i, j) for i in range(_C) for j in range(i + 1, _C)]
_DIM = len(_PAIRS)
_NC, _NS, _L = 2, 16, 16
_NW = _NC * _NS
_NPAR = _C + 3 + _DIM


@functools.lru_cache(maxsize=None)
def _build_sc_kernel(B: int):
    rows_per_w = B // _NW
    elems_per_w = rows_per_w * _C
    n_groups = rows_per_w // _L
    mesh = plsc.VectorSubcoreMesh(
        core_axis_name="c", subcore_axis_name="s",
        num_cores=_NC, num_subcores=_NS)

    @functools.partial(
        pl.kernel,
        mesh=mesh,
        compiler_params=pltpu.CompilerParams(needs_layout_passes=False),
        out_type=jax.ShapeDtypeStruct((B * _C,), jnp.float32),
        scratch_types=[
            pltpu.VMEM((elems_per_w,), jnp.float32),
            pltpu.VMEM((elems_per_w,), jnp.float32),
            pltpu.VMEM((_NPAR, _L), jnp.float32),
        ],
    )
    def k(x_hbm, par_hbm, out_hbm, xv, ov, pv):
        wid = lax.axis_index("s") * _NC + lax.axis_index("c")
        base = wid * elems_per_w
        pltpu.sync_copy(x_hbm.at[pl.ds(base, elems_per_w)], xv)
        pltpu.sync_copy(par_hbm, pv)

        riota = lax.broadcasted_iota(jnp.int32, (_L,), 0) * _C
        d1 = [pv[c, :] for c in range(_C)]
        w0, w1, w2v = pv[_C, :], pv[_C + 1, :], pv[_C + 2, :]

        def body(g, carry):
            off = g * (_L * _C)
            ch = []
            for c in range(_C):
                idx = riota + (off + c)
                ch.append(plsc.load_gather(xv, [idx]) * d1[c])
            acc = [None] * _C
            for p, (i, j) in enumerate(_PAIRS):
                d = ch[i] - ch[j]
                a = jnp.abs(d)
                s = jnp.sign(d)
                a2 = a * a
                v = a * w0 + a2 * w1 + a2 * a * w2v
                v = jnp.where(v >= 0, v, v * jnp.float32(0.01))
                t = v * s * pv[_C + 3 + p, :]
                acc[i] = (-t) if acc[i] is None else (acc[i] - t)
                acc[j] = t if acc[j] is None else (acc[j] + t)
            for c in range(_C):
                plsc.store_scatter(ov, [riota + (off + c)], acc[c])
            return carry

        lax.fori_loop(0, n_groups, body, 0, unroll=False)
        pltpu.sync_copy(ov, out_hbm.at[pl.ds(base, elems_per_w)])

    return k


def kernel(x, diag1, w2, diag3, diff_indices, i_idx, j_idx):
    B = x.shape[0]
    par = jnp.concatenate([diag1.astype(jnp.float32),
                           w2[0].astype(jnp.float32),
                           diag3.astype(jnp.float32)])
    par = jnp.broadcast_to(par[:, None], (_NPAR, _L))
    out = _build_sc_kernel(B)(x.reshape(B * _C), par)
    return out.reshape(B, _C, 1)


# speedup vs baseline: 4.4194x; 1.0308x over previous
import functools

import jax
import jax.numpy as jnp
from jax import lax
from jax.experimental import pallas as pl
from jax.experimental.pallas import tpu as pltpu
from jax.experimental.pallas import tpu_sc as plsc

_C = 8
_PAIRS = [(i, j) for i in range(_C) for j in range(i + 1, _C)]
_DIM = len(_PAIRS)
_NC, _NS, _L = 2, 16, 16
_NW = _NC * _NS
_NPAR = _C + 3 + _DIM
_GW = _L * _C


@functools.lru_cache(maxsize=None)
def _build_sc_kernel(B: int):
    n_groups_total = B // _L
    groups_per_w = n_groups_total // _NW
    mesh = plsc.VectorSubcoreMesh(
        core_axis_name="c", subcore_axis_name="s",
        num_cores=_NC, num_subcores=_NS)

    @functools.partial(
        pl.kernel,
        mesh=mesh,
        compiler_params=pltpu.CompilerParams(needs_layout_passes=False),
        out_type=jax.ShapeDtypeStruct((n_groups_total, _GW), jnp.float32),
        scratch_types=[
            pltpu.VMEM((groups_per_w, _GW), jnp.float32),
            pltpu.VMEM((groups_per_w, _GW), jnp.float32),
            pltpu.VMEM((_NPAR, _L), jnp.float32),
        ],
    )
    def k(x_hbm, par_hbm, out_hbm, xv, ov, pv):
        wid = lax.axis_index("s") * _NC + lax.axis_index("c")
        g0 = wid * groups_per_w
        pltpu.sync_copy(x_hbm.at[pl.ds(g0, groups_per_w)], xv)
        pltpu.sync_copy(par_hbm, pv)

        riota = lax.broadcasted_iota(jnp.int32, (_L,), 0) * _C
        rc = [riota + c for c in range(_C)]
        d1 = [pv[c, :] for c in range(_C)]
        w0, w1, w2v = pv[_C, :], pv[_C + 1, :], pv[_C + 2, :]
        sign_bit = jnp.full((_L,), jnp.int32(-2147483648))

        def body(g, carry):
            gidx = jnp.zeros((_L,), jnp.int32) + g
            ch = []
            for c in range(_C):
                ch.append(plsc.load_gather(xv, [gidx, rc[c]]) * d1[c])
            acc = [None] * _C
            for p, (i, j) in enumerate(_PAIRS):
                d = ch[i] - ch[j]
                a = jnp.abs(d)
                sb = plsc.bitcast(d, jnp.int32) & sign_bit
                a2 = a * a
                v = a * w0 + a2 * w1 + a2 * a * w2v
                v = jnp.maximum(v, v * jnp.float32(0.01))
                v = v * pv[_C + 3 + p, :]
                t = plsc.bitcast(plsc.bitcast(v, jnp.int32) ^ sb, jnp.float32)
                acc[i] = (-t) if acc[i] is None else (acc[i] - t)
                acc[j] = t if acc[j] is None else (acc[j] + t)
            for c in range(_C):
                plsc.store_scatter(ov, [gidx, rc[c]], acc[c])
            return carry

        lax.fori_loop(0, groups_per_w, body, 0, unroll=False)
        pltpu.sync_copy(ov, out_hbm.at[pl.ds(g0, groups_per_w)])

    return k


def kernel(x, diag1, w2, diag3, diff_indices, i_idx, j_idx):
    B = x.shape[0]
    par = jnp.concatenate([diag1.astype(jnp.float32),
                           w2[0].astype(jnp.float32),
                           diag3.astype(jnp.float32)])
    par = jnp.broadcast_to(par[:, None], (_NPAR, _L))
    out = _build_sc_kernel(B)(x.reshape(B // _L, _L * _C), par)
    return out.reshape(B, _C, 1)


# speedup vs baseline: 5.5556x; 1.2571x over previous
import functools

import jax
import jax.numpy as jnp
from jax import lax
from jax.experimental import pallas as pl
from jax.experimental.pallas import tpu as pltpu
from jax.experimental.pallas import tpu_sc as plsc

_C = 8
_PAIRS = [(i, j) for i in range(_C) for j in range(i + 1, _C)]
_DIM = len(_PAIRS)
_NC, _NS, _L = 2, 16, 16
_NW = _NC * _NS
_NPAR = _C + 3 + _DIM
_GW = _L * _C


@functools.lru_cache(maxsize=None)
def _build_sc_kernel(B: int):
    n_groups_total = B // _L
    groups_per_w = n_groups_total // _NW
    mesh = plsc.VectorSubcoreMesh(
        core_axis_name="c", subcore_axis_name="s",
        num_cores=_NC, num_subcores=_NS)

    rows_per_w = B // _NW

    @functools.partial(
        pl.kernel,
        mesh=mesh,
        compiler_params=pltpu.CompilerParams(needs_layout_passes=False),
        out_type=jax.ShapeDtypeStruct((_C, B), jnp.float32),
        scratch_types=[
            pltpu.VMEM((groups_per_w, _GW), jnp.float32),
            pltpu.VMEM((_C, rows_per_w), jnp.float32),
            pltpu.VMEM((_NPAR, _L), jnp.float32),
        ],
    )
    def k(x_hbm, par_hbm, out_hbm, xv, ov, pv):
        wid = lax.axis_index("s") * _NC + lax.axis_index("c")
        g0 = wid * groups_per_w
        b0 = wid * rows_per_w
        pltpu.sync_copy(x_hbm.at[pl.ds(g0, groups_per_w)], xv)
        pltpu.sync_copy(par_hbm, pv)

        riota = lax.broadcasted_iota(jnp.int32, (_L,), 0) * _C
        rc = [riota + c for c in range(_C)]
        d1 = [pv[c, :] for c in range(_C)]
        w0, w1, w2v = pv[_C, :], pv[_C + 1, :], pv[_C + 2, :]
        sign_bit = jnp.full((_L,), jnp.int32(-2147483648))

        def body(g, carry):
            gidx = jnp.zeros((_L,), jnp.int32) + g
            ch = []
            for c in range(_C):
                ch.append(plsc.load_gather(xv, [gidx, rc[c]]) * d1[c])
            acc = [None] * _C
            for p, (i, j) in enumerate(_PAIRS):
                d = ch[i] - ch[j]
                a = jnp.abs(d)
                sb = plsc.bitcast(d, jnp.int32) & sign_bit
                a2 = a * a
                v = a * w0 + a2 * w1 + a2 * a * w2v
                v = jnp.maximum(v, v * jnp.float32(0.01))
                v = v * pv[_C + 3 + p, :]
                t = plsc.bitcast(plsc.bitcast(v, jnp.int32) ^ sb, jnp.float32)
                acc[i] = (-t) if acc[i] is None else (acc[i] - t)
                acc[j] = t if acc[j] is None else (acc[j] + t)
            for c in range(_C):
                ov[c, pl.ds(g * _L, _L)] = acc[c]
            return carry

        lax.fori_loop(0, groups_per_w, body, 0, unroll=False)
        for c in range(_C):
            pltpu.sync_copy(ov.at[c], out_hbm.at[c, pl.ds(b0, rows_per_w)])

    return k


def kernel(x, diag1, w2, diag3, diff_indices, i_idx, j_idx):
    B = x.shape[0]
    par = jnp.concatenate([diag1.astype(jnp.float32),
                           w2[0].astype(jnp.float32),
                           diag3.astype(jnp.float32)])
    par = jnp.broadcast_to(par[:, None], (_NPAR, _L))
    out = _build_sc_kernel(B)(x.reshape(B // _L, _L * _C), par)
    return out.T[:, :, None]


# speedup vs baseline: 6.7690x; 1.2184x over previous
import functools

import jax
import jax.numpy as jnp
from jax import lax
from jax.experimental import pallas as pl
from jax.experimental.pallas import tpu as pltpu
from jax.experimental.pallas import tpu_sc as plsc

_C = 8
_PAIRS = [(i, j) for i in range(_C) for j in range(i + 1, _C)]
_DIM = len(_PAIRS)
_NC, _NS, _L = 2, 16, 16
_NW = _NC * _NS
_NPAR = _C + 3 + _DIM
_GW = _L * _C


@functools.lru_cache(maxsize=None)
def _build_sc_kernel(B: int):
    n_groups_total = B // _L
    groups_per_w = n_groups_total // _NW
    mesh = plsc.VectorSubcoreMesh(
        core_axis_name="c", subcore_axis_name="s",
        num_cores=_NC, num_subcores=_NS)

    rows_per_w = B // _NW

    @functools.partial(
        pl.kernel,
        mesh=mesh,
        compiler_params=pltpu.CompilerParams(needs_layout_passes=False),
        out_type=jax.ShapeDtypeStruct((_C, B), jnp.float32),
        scratch_types=[
            pltpu.VMEM((_C, rows_per_w), jnp.float32),
            pltpu.VMEM((_C, rows_per_w), jnp.float32),
            pltpu.VMEM((_NPAR, _L), jnp.float32),
        ],
    )
    def k(x_hbm, par_hbm, out_hbm, xv, ov, pv):
        wid = lax.axis_index("s") * _NC + lax.axis_index("c")
        b0 = wid * rows_per_w
        for c in range(_C):
            pltpu.sync_copy(x_hbm.at[c, pl.ds(b0, rows_per_w)], xv.at[c])
        pltpu.sync_copy(par_hbm, pv)

        d1 = [pv[c, :] for c in range(_C)]
        w0, w1, w2v = pv[_C, :], pv[_C + 1, :], pv[_C + 2, :]
        sign_bit = jnp.full((_L,), jnp.int32(-2147483648))

        def body(g, carry):
            ch = []
            for c in range(_C):
                ch.append(xv[c, pl.ds(g * _L, _L)] * d1[c])
            acc = [None] * _C
            for p, (i, j) in enumerate(_PAIRS):
                d = ch[i] - ch[j]
                a = jnp.abs(d)
                sb = plsc.bitcast(d, jnp.int32) & sign_bit
                a2 = a * a
                v = a * w0 + a2 * w1 + a2 * a * w2v
                v = jnp.maximum(v, v * jnp.float32(0.01))
                v = v * pv[_C + 3 + p, :]
                t = plsc.bitcast(plsc.bitcast(v, jnp.int32) ^ sb, jnp.float32)
                acc[i] = (-t) if acc[i] is None else (acc[i] - t)
                acc[j] = t if acc[j] is None else (acc[j] + t)
            for c in range(_C):
                ov[c, pl.ds(g * _L, _L)] = acc[c]
            return carry

        lax.fori_loop(0, groups_per_w, body, 0, unroll=False)
        for c in range(_C):
            pltpu.sync_copy(ov.at[c], out_hbm.at[c, pl.ds(b0, rows_per_w)])

    return k


def kernel(x, diag1, w2, diag3, diff_indices, i_idx, j_idx):
    B = x.shape[0]
    par = jnp.concatenate([diag1.astype(jnp.float32),
                           w2[0].astype(jnp.float32),
                           diag3.astype(jnp.float32)])
    par = jnp.broadcast_to(par[:, None], (_NPAR, _L))
    out = _build_sc_kernel(B)(x.T, par)
    return out.T[:, :, None]


# speedup vs baseline: 8.1356x; 1.2019x over previous
import functools

import jax
import jax.numpy as jnp
from jax import lax
from jax.experimental import pallas as pl
from jax.experimental.pallas import tpu as pltpu
from jax.experimental.pallas import tpu_sc as plsc

_C = 8
_PAIRS = [(i, j) for i in range(_C) for j in range(i + 1, _C)]
_DIM = len(_PAIRS)
_NC, _NS, _L = 2, 16, 16
_NW = _NC * _NS
_NPAR = _C + 3 + _DIM
_GW = _L * _C


@functools.lru_cache(maxsize=None)
def _build_sc_kernel(B: int):
    n_groups_total = B // _L
    groups_per_w = n_groups_total // _NW
    mesh = plsc.VectorSubcoreMesh(
        core_axis_name="c", subcore_axis_name="s",
        num_cores=_NC, num_subcores=_NS)

    rows_per_w = B // _NW

    @functools.partial(
        pl.kernel,
        mesh=mesh,
        compiler_params=pltpu.CompilerParams(needs_layout_passes=False),
        out_type=jax.ShapeDtypeStruct((_C, B), jnp.float32),
        scratch_types=[
            pltpu.VMEM((_C, rows_per_w), jnp.float32),
            pltpu.VMEM((_C, rows_per_w), jnp.float32),
            pltpu.VMEM((_NPAR, _L), jnp.float32),
        ],
    )
    def k(x_hbm, par_hbm, out_hbm, xv, ov, pv):
        wid = lax.axis_index("s") * _NC + lax.axis_index("c")
        b0 = wid * rows_per_w
        pltpu.sync_copy(x_hbm.at[:, pl.ds(b0, rows_per_w)], xv)
        pltpu.sync_copy(par_hbm, pv)

        d1 = [pv[c, :] for c in range(_C)]
        w0, w1, w2v = pv[_C, :], pv[_C + 1, :], pv[_C + 2, :]
        sign_bit = jnp.full((_L,), jnp.int32(-2147483648))

        def body(g, carry):
            ch = []
            for c in range(_C):
                ch.append(xv[c, pl.ds(g * _L, _L)] * d1[c])
            acc = [None] * _C
            for p, (i, j) in enumerate(_PAIRS):
                d = ch[i] - ch[j]
                a = jnp.abs(d)
                sb = plsc.bitcast(d, jnp.int32) & sign_bit
                v = a * (w0 + a * (w1 + a * w2v))
                v = jnp.maximum(v, v * jnp.float32(0.01))
                v = v * pv[_C + 3 + p, :]
                t = plsc.bitcast(plsc.bitcast(v, jnp.int32) ^ sb, jnp.float32)
                acc[i] = (-t) if acc[i] is None else (acc[i] - t)
                acc[j] = t if acc[j] is None else (acc[j] + t)
            for c in range(_C):
                ov[c, pl.ds(g * _L, _L)] = acc[c]
            return carry

        lax.fori_loop(0, groups_per_w, body, 0, unroll=False)
        pltpu.sync_copy(ov, out_hbm.at[:, pl.ds(b0, rows_per_w)])

    return k


def kernel(x, diag1, w2, diag3, diff_indices, i_idx, j_idx):
    B = x.shape[0]
    par = jnp.concatenate([diag1.astype(jnp.float32),
                           w2[0].astype(jnp.float32),
                           diag3.astype(jnp.float32)])
    par = jnp.broadcast_to(par[:, None], (_NPAR, _L))
    out = _build_sc_kernel(B)(x.T, par)
    return out.T[:, :, None]
